# Initial kernel scaffold; baseline (speedup 1.0000x reference)
import functools

import jax
import jax.numpy as jnp
from jax import lax
from jax.experimental import pallas as pl
from jax.experimental.pallas import tpu as pltpu
from jax.experimental.pallas import tpu_sc as plsc

_N = 320000
_D = 128
_C = 10000

_NC = 2
_NS = 16
_L = 16

_CHUNK = 80
_ROWS_PER_TILE = _N // _NS
_NCHUNKS = _ROWS_PER_TILE // _CHUNK
_CSTRIPE = _C // _NS
_ZROWS = 125
_NZ = _CSTRIPE // _ZROWS


def _fill_rows(ref, nrows, ncols, value):
    def body(r, _):
        for c in range(ncols // _L):
            ref[r, pl.ds(c * _L, _L)] = jnp.full((_L,), value, jnp.float32)
        return 0
    lax.fori_loop(0, nrows, body, 0)


def _sc_accumulate(embeddings, labels):
    mesh = plsc.VectorSubcoreMesh(core_axis_name="c", subcore_axis_name="s")

    @functools.partial(
        pl.kernel,
        mesh=mesh,
        out_type=[
            jax.ShapeDtypeStruct((_C, _L), jnp.float32),
            jax.ShapeDtypeStruct((_C, _D), jnp.float32),
            jax.ShapeDtypeStruct((_C, _D), jnp.float32),
        ],
        scratch_types=[
            pltpu.VMEM((_CHUNK, _D), jnp.float32),
            pltpu.VMEM((_CHUNK,), jnp.int32),
            pltpu.VMEM((_CHUNK, _L), jnp.float32),
            pltpu.VMEM((_ZROWS, _D), jnp.float32),
            pltpu.VMEM((_CSTRIPE, _L), jnp.float32),
            pltpu.VMEM_SHARED((_C, _D), jnp.float32),
            pltpu.VMEM_SHARED((_C, _L), jnp.float32),
        ],
    )
    def sc_kernel(emb_hbm, lbl_hbm, cnt_out, sum_out, sq_out,
                  row_v, lbl_v, ones_v, zrow_v, zcnt_v, acc_sh, cnt_sh):
        core = lax.axis_index("c")
        tile = lax.axis_index("s")

        _fill_rows(zrow_v, _ZROWS, _D, 0.0)
        crow = tile * _CSTRIPE
        for z in range(_NZ):
            pltpu.sync_copy(zrow_v, acc_sh.at[pl.ds(crow + z * _ZROWS, _ZROWS)])

        @pl.when(core == 0)
        def _():
            _fill_rows(ones_v, _CHUNK, _L, 1.0)
            _fill_rows(zcnt_v, _CSTRIPE, _L, 0.0)
            pltpu.sync_copy(zcnt_v, cnt_sh.at[pl.ds(crow, _CSTRIPE)])

        plsc.subcore_barrier()

        base0 = tile * _ROWS_PER_TILE

        @pl.when(core == 0)
        def _():
            def body(k, _):
                base = base0 + k * _CHUNK
                pltpu.sync_copy(lbl_hbm.at[pl.ds(base, _CHUNK)], lbl_v)
                pltpu.sync_copy(emb_hbm.at[pl.ds(base, _CHUNK)], row_v)
                pltpu.sync_copy(row_v, acc_sh.at[lbl_v], add=True)
                pltpu.sync_copy(ones_v, cnt_sh.at[lbl_v], add=True)
                return 0
            lax.fori_loop(0, _NCHUNKS, body, 0)

        @pl.when(core == 1)
        def _():
            def body(k, _):
                base = base0 + k * _CHUNK
                pltpu.sync_copy(lbl_hbm.at[pl.ds(base, _CHUNK)], lbl_v)
                pltpu.sync_copy(emb_hbm.at[pl.ds(base, _CHUNK)], row_v)

                def sq(r, _):
                    for c in range(_D // _L):
                        v = row_v[r, pl.ds(c * _L, _L)]
                        row_v[r, pl.ds(c * _L, _L)] = v * v
                    return 0
                lax.fori_loop(0, _CHUNK, sq, 0)
                pltpu.sync_copy(row_v, acc_sh.at[lbl_v], add=True)
                return 0
            lax.fori_loop(0, _NCHUNKS, body, 0)

        plsc.subcore_barrier()

        @pl.when(core == 0)
        def _():
            for z in range(_NZ):
                r0 = crow + z * _ZROWS
                pltpu.sync_copy(acc_sh.at[pl.ds(r0, _ZROWS)],
                                sum_out.at[pl.ds(r0, _ZROWS)])
            pltpu.sync_copy(cnt_sh.at[pl.ds(crow, _CSTRIPE)],
                            cnt_out.at[pl.ds(crow, _CSTRIPE)])

        @pl.when(core == 1)
        def _():
            for z in range(_NZ):
                r0 = crow + z * _ZROWS
                pltpu.sync_copy(acc_sh.at[pl.ds(r0, _ZROWS)],
                                sq_out.at[pl.ds(r0, _ZROWS)])

    return sc_kernel(embeddings, labels)


def _finalize_body(cnt_ref, sum_ref, sq_ref, mean_ref, std_ref, om_ref, os_ref):
    n = cnt_ref[:, 0:1]
    safe_n = jnp.maximum(n, 1.0)
    mean = sum_ref[...] / safe_n
    denom = jnp.maximum(n - 1.0, 1.0)
    var = (sq_ref[...] - safe_n * mean * mean) / denom
    std = jnp.sqrt(jnp.maximum(var, 0.0))
    present = n > 0.0
    om_ref[...] = jnp.where(present, mean, mean_ref[...])
    os_ref[...] = jnp.where(present, std, std_ref[...])


def kernel(embeddings, labels, means, stds):
    counts, sums, sumsq = _sc_accumulate(embeddings, labels)

    blk = 1000
    grid = _C // blk
    row_spec = pl.BlockSpec((blk, _D), lambda i: (i, 0))
    new_means, new_stds = pl.pallas_call(
        _finalize_body,
        grid=(grid,),
        in_specs=[pl.BlockSpec((blk, _L), lambda i: (i, 0)),
                  row_spec, row_spec, row_spec, row_spec],
        out_specs=[row_spec, row_spec],
        out_shape=[jax.ShapeDtypeStruct((_C, _D), jnp.float32)] * 2,
    )(counts, sums, sumsq, means, stds)
    return new_means, new_stds


# speedup vs baseline: 2.8497x; 2.8497x over previous
import functools

import jax
import jax.numpy as jnp
from jax import lax
from jax.experimental import pallas as pl
from jax.experimental.pallas import tpu as pltpu
from jax.experimental.pallas import tpu_sc as plsc

_N = 320000
_D = 128
_C = 10000

_NS = 16
_L = 16

_CHUNK = 80
_ROWS_PER_TILE = _N // _NS
_NCHUNKS = _ROWS_PER_TILE // _CHUNK
_CROWS_PER_TILE = _N // 2 // _NS
_CNCHUNKS = _CROWS_PER_TILE // _CHUNK

_CSTRIPE = 640
_ZROWS = 80
_NZ_FULL = _CSTRIPE // _ZROWS
_NZ_LAST = (_C - 15 * _CSTRIPE) // _ZROWS


def _over_stripe(tile, fn):
    @pl.when(tile < _NS - 1)
    def _():
        for z in range(_NZ_FULL):
            fn(tile * _CSTRIPE + z * _ZROWS)

    @pl.when(tile == _NS - 1)
    def _():
        for z in range(_NZ_LAST):
            fn((_NS - 1) * _CSTRIPE + z * _ZROWS)


def _zero_arith(ref, seed_ref):
    def zbody(r, _):
        for c in range(_D // _L):
            v = seed_ref[r, pl.ds(c * _L, _L)]
            ref[r, pl.ds(c * _L, _L)] = v * 0.0
        return 0
    lax.fori_loop(0, _CHUNK, zbody, 0)


def _sc_sums(embeddings, labels):
    mesh = plsc.VectorSubcoreMesh(core_axis_name="c", subcore_axis_name="s")

    @functools.partial(
        pl.kernel,
        mesh=mesh,
        out_type=jax.ShapeDtypeStruct((2 * _C, _D), jnp.float32),
        scratch_types=[
            pltpu.VMEM((_CHUNK, _D), jnp.float32),
            pltpu.VMEM((_CHUNK,), jnp.int32),
            pltpu.VMEM_SHARED((_C, _D), jnp.float32),
        ],
    )
    def sc_kernel(emb_hbm, lbl_hbm, acc_out, row_v, lbl_v, acc_sh):
        core = lax.axis_index("c")
        tile = lax.axis_index("s")
        fcore = lax.convert_element_type(core, jnp.float32)

        pltpu.sync_copy(emb_hbm.at[pl.ds(0, _CHUNK)], row_v)
        _zero_arith(row_v, row_v)
        _over_stripe(tile, lambda r0: pltpu.sync_copy(
            row_v, acc_sh.at[pl.ds(r0, _ZROWS)]))

        plsc.subcore_barrier()

        base0 = tile * _ROWS_PER_TILE

        def body(k, _):
            base = base0 + k * _CHUNK
            pltpu.sync_copy(lbl_hbm.at[pl.ds(base, _CHUNK)], lbl_v)
            pltpu.sync_copy(emb_hbm.at[pl.ds(base, _CHUNK)], row_v)

            def sq(r, _):
                for c in range(_D // _L):
                    v = row_v[r, pl.ds(c * _L, _L)]
                    row_v[r, pl.ds(c * _L, _L)] = v * (fcore * v +
                                                       (1.0 - fcore))
                return 0
            lax.fori_loop(0, _CHUNK, sq, 0)
            pltpu.sync_copy(row_v, acc_sh.at[lbl_v], add=True)
            return 0
        lax.fori_loop(0, _NCHUNKS, body, 0)

        plsc.subcore_barrier()

        _over_stripe(tile, lambda r0: pltpu.sync_copy(
            acc_sh.at[pl.ds(r0, _ZROWS)],
            acc_out.at[pl.ds(core * _C + r0, _ZROWS)]))

    return sc_kernel(embeddings, labels)


def _sc_counts(embeddings, labels):
    mesh = plsc.VectorSubcoreMesh(core_axis_name="c", subcore_axis_name="s")

    @functools.partial(
        pl.kernel,
        mesh=mesh,
        out_type=jax.ShapeDtypeStruct((2 * _C, _D), jnp.float32),
        scratch_types=[
            pltpu.VMEM((_CHUNK, _D), jnp.float32),
            pltpu.VMEM((_CHUNK,), jnp.int32),
            pltpu.VMEM_SHARED((_C, _D), jnp.float32),
        ],
    )
    def sc_kernel(emb_hbm, lbl_hbm, cnt_out, ones_v, lbl_v, cnt_sh):
        core = lax.axis_index("c")
        tile = lax.axis_index("s")

        pltpu.sync_copy(emb_hbm.at[pl.ds(0, _CHUNK)], ones_v)
        _zero_arith(ones_v, ones_v)
        _over_stripe(tile, lambda r0: pltpu.sync_copy(
            ones_v, cnt_sh.at[pl.ds(r0, _ZROWS)]))

        def obody(r, _):
            for c in range(_D // _L):
                z = ones_v[r, pl.ds(c * _L, _L)]
                ones_v[r, pl.ds(c * _L, _L)] = z + 1.0
            return 0
        lax.fori_loop(0, _CHUNK, obody, 0)

        plsc.subcore_barrier()

        base0 = (core * _NS + tile) * _CROWS_PER_TILE

        def body(k, _):
            base = base0 + k * _CHUNK
            pltpu.sync_copy(lbl_hbm.at[pl.ds(base, _CHUNK)], lbl_v)
            pltpu.sync_copy(ones_v, cnt_sh.at[lbl_v], add=True)
            return 0
        lax.fori_loop(0, _CNCHUNKS, body, 0)

        plsc.subcore_barrier()

        _over_stripe(tile, lambda r0: pltpu.sync_copy(
            cnt_sh.at[pl.ds(r0, _ZROWS)],
            cnt_out.at[pl.ds(core * _C + r0, _ZROWS)]))

    return sc_kernel(embeddings, labels)


def _finalize_body(cnta_ref, cntb_ref, sum_ref, sq_ref, mean_ref, std_ref,
                   om_ref, os_ref):
    n = cnta_ref[:, 0:1] + cntb_ref[:, 0:1]
    safe_n = jnp.maximum(n, 1.0)
    mean = sum_ref[...] / safe_n
    denom = jnp.maximum(n - 1.0, 1.0)
    var = (sq_ref[...] - safe_n * mean * mean) / denom
    std = jnp.sqrt(jnp.maximum(var, 0.0))
    present = n > 0.0
    om_ref[...] = jnp.where(present, mean, mean_ref[...])
    os_ref[...] = jnp.where(present, std, std_ref[...])


def kernel(embeddings, labels, means, stds):
    acc = _sc_sums(embeddings, labels)
    cnt2 = _sc_counts(embeddings, labels)
    sums = acc[:_C]
    sumsq = acc[_C:]

    blk = 1000
    spec = pl.BlockSpec((blk, _D), lambda i: (i, 0))
    new_means, new_stds = pl.pallas_call(
        _finalize_body,
        grid=(_C // blk,),
        in_specs=[spec] * 6,
        out_specs=[spec, spec],
        out_shape=[jax.ShapeDtypeStruct((_C, _D), jnp.float32)] * 2,
    )(cnt2[:_C], cnt2[_C:], sums, sumsq, means, stds)
    return new_means, new_stds


# speedup vs baseline: 6.3195x; 2.2176x over previous
import functools

import jax
import jax.numpy as jnp
from jax import lax
from jax.experimental import pallas as pl
from jax.experimental.pallas import tpu as pltpu
from jax.experimental.pallas import tpu_sc as plsc

_N = 320000
_D = 128
_C = 10000

_NS = 16
_L = 16

_CHUNK = 80
_ROWS_PER_TILE = _N // _NS
_NCHUNKS = _ROWS_PER_TILE // _CHUNK
_CROWS_PER_TILE = _N // 2 // _NS
_CNCHUNKS = _CROWS_PER_TILE // _CHUNK

_CSTRIPE = 640
_ZROWS = 80
_NZ_FULL = _CSTRIPE // _ZROWS
_NZ_LAST = (_C - 15 * _CSTRIPE) // _ZROWS


def _over_stripe(tile, fn):
    @pl.when(tile < _NS - 1)
    def _():
        for z in range(_NZ_FULL):
            fn(tile * _CSTRIPE + z * _ZROWS)

    @pl.when(tile == _NS - 1)
    def _():
        for z in range(_NZ_LAST):
            fn((_NS - 1) * _CSTRIPE + z * _ZROWS)


def _zero_arith(ref, seed_ref):
    def zbody(r, _):
        for c in range(_D // _L):
            v = seed_ref[r, pl.ds(c * _L, _L)]
            ref[r, pl.ds(c * _L, _L)] = v * 0.0
        return 0
    lax.fori_loop(0, _CHUNK, zbody, 0)


def _sc_sums(embeddings, labels):
    mesh = plsc.VectorSubcoreMesh(core_axis_name="c", subcore_axis_name="s")

    @functools.partial(
        pl.kernel,
        mesh=mesh,
        out_type=jax.ShapeDtypeStruct((2 * _C, _D), jnp.float32),
        scratch_types=[
            pltpu.VMEM((_CHUNK, _D), jnp.float32),
            pltpu.VMEM((_CHUNK, _D), jnp.float32),
            pltpu.VMEM((_CHUNK, _D), jnp.float32),
            pltpu.VMEM((_CHUNK, _D), jnp.float32),
            pltpu.VMEM((_CHUNK,), jnp.int32),
            pltpu.VMEM((_CHUNK,), jnp.int32),
            pltpu.VMEM((_CHUNK,), jnp.int32),
            pltpu.VMEM((_CHUNK,), jnp.int32),
            pltpu.SemaphoreType.DMA,
            pltpu.SemaphoreType.DMA,
            pltpu.SemaphoreType.DMA,
            pltpu.SemaphoreType.DMA,
            pltpu.SemaphoreType.DMA,
            pltpu.SemaphoreType.DMA,
            pltpu.SemaphoreType.DMA,
            pltpu.SemaphoreType.DMA,
            pltpu.VMEM_SHARED((_C, _D), jnp.float32),
        ],
    )
    def sc_kernel(emb_hbm, lbl_hbm, acc_out,
                  row0, row1, row2, row3, lb0, lb1, lb2, lb3,
                  ls0, ls1, ls2, ls3, ss0, ss1, ss2, ss3, acc_sh):
        rows = [row0, row1, row2, row3]
        lbls = [lb0, lb1, lb2, lb3]
        lsem = [ls0, ls1, ls2, ls3]
        ssem = [ss0, ss1, ss2, ss3]

        core = lax.axis_index("c")
        tile = lax.axis_index("s")
        fcore = lax.convert_element_type(core, jnp.float32)
        base0 = tile * _ROWS_PER_TILE

        pltpu.sync_copy(emb_hbm.at[pl.ds(0, _CHUNK)], row0)
        _zero_arith(row0, row0)
        _over_stripe(tile, lambda r0: pltpu.sync_copy(
            row0, acc_sh.at[pl.ds(r0, _ZROWS)]))

        plsc.subcore_barrier()

        def issue_loads(j, b):
            base = base0 + j * _CHUNK
            pltpu.async_copy(lbl_hbm.at[pl.ds(base, _CHUNK)], lbls[b],
                             lsem[b])
            pltpu.async_copy(emb_hbm.at[pl.ds(base, _CHUNK)], rows[b],
                             lsem[b])

        def wait_loads(b):
            pltpu.make_async_copy(lbl_hbm.at[pl.ds(0, _CHUNK)], lbls[b],
                                  lsem[b]).wait()
            pltpu.make_async_copy(emb_hbm.at[pl.ds(0, _CHUNK)], rows[b],
                                  lsem[b]).wait()

        def issue_scatter(b):
            pltpu.async_copy(rows[b], acc_sh.at[lbls[b]], ssem[b], add=True)

        def wait_scatter(b):
            pltpu.make_async_copy(rows[b], acc_sh.at[lbls[b]],
                                  ssem[b]).wait()

        def squares(b):
            def sq(r, _):
                for c in range(_D // _L):
                    v = rows[b][r, pl.ds(c * _L, _L)]
                    rows[b][r, pl.ds(c * _L, _L)] = v * (fcore * v +
                                                         (1.0 - fcore))
                return 0
            lax.fori_loop(0, _CHUNK, sq, 0)

        issue_loads(0, 0)
        issue_loads(1, 1)

        def chunk_body(j, b):
            wait_loads(b)
            squares(b)
            issue_scatter(b)
            b2 = (b + 2) % 4

            @pl.when(j >= 2)
            def _():
                wait_scatter(b2)
            issue_loads(j + 2, b2)

        def gbody(g, _):
            for i in range(4):
                chunk_body(4 * g + i, i)
            return 0
        lax.fori_loop(0, (_NCHUNKS - 2) // 4, gbody, 0)

        for (j, b) in ((_NCHUNKS - 2, 0), (_NCHUNKS - 1, 1)):
            wait_loads(b)
            squares(b)
            issue_scatter(b)

        for b in range(4):
            wait_scatter(b)

        plsc.subcore_barrier()

        _over_stripe(tile, lambda r0: pltpu.sync_copy(
            acc_sh.at[pl.ds(r0, _ZROWS)],
            acc_out.at[pl.ds(core * _C + r0, _ZROWS)]))

    return sc_kernel(embeddings, labels)


def _sc_counts(embeddings, labels):
    mesh = plsc.VectorSubcoreMesh(core_axis_name="c", subcore_axis_name="s")

    @functools.partial(
        pl.kernel,
        mesh=mesh,
        out_type=jax.ShapeDtypeStruct((2 * _C, _D), jnp.float32),
        scratch_types=[
            pltpu.VMEM((_CHUNK, _D), jnp.float32),
            pltpu.VMEM((_CHUNK,), jnp.int32),
            pltpu.VMEM((_CHUNK,), jnp.int32),
            pltpu.VMEM((_CHUNK,), jnp.int32),
            pltpu.VMEM((_CHUNK,), jnp.int32),
            pltpu.SemaphoreType.DMA,
            pltpu.SemaphoreType.DMA,
            pltpu.SemaphoreType.DMA,
            pltpu.SemaphoreType.DMA,
            pltpu.SemaphoreType.DMA,
            pltpu.SemaphoreType.DMA,
            pltpu.SemaphoreType.DMA,
            pltpu.SemaphoreType.DMA,
            pltpu.VMEM_SHARED((_C, _D), jnp.float32),
        ],
    )
    def sc_kernel(emb_hbm, lbl_hbm, cnt_out, ones_v,
                  lb0, lb1, lb2, lb3, ls0, ls1, ls2, ls3,
                  ss0, ss1, ss2, ss3, cnt_sh):
        lbls = [lb0, lb1, lb2, lb3]
        lsem = [ls0, ls1, ls2, ls3]
        ssem = [ss0, ss1, ss2, ss3]

        core = lax.axis_index("c")
        tile = lax.axis_index("s")
        base0 = (core * _NS + tile) * _CROWS_PER_TILE

        pltpu.sync_copy(emb_hbm.at[pl.ds(0, _CHUNK)], ones_v)
        _zero_arith(ones_v, ones_v)
        _over_stripe(tile, lambda r0: pltpu.sync_copy(
            ones_v, cnt_sh.at[pl.ds(r0, _ZROWS)]))

        def obody(r, _):
            for c in range(_D // _L):
                z = ones_v[r, pl.ds(c * _L, _L)]
                ones_v[r, pl.ds(c * _L, _L)] = z + 1.0
            return 0
        lax.fori_loop(0, _CHUNK, obody, 0)

        plsc.subcore_barrier()

        def issue_load(j, b):
            base = base0 + j * _CHUNK
            pltpu.async_copy(lbl_hbm.at[pl.ds(base, _CHUNK)], lbls[b],
                             lsem[b])

        def wait_load(b):
            pltpu.make_async_copy(lbl_hbm.at[pl.ds(0, _CHUNK)], lbls[b],
                                  lsem[b]).wait()

        def issue_scatter(b):
            pltpu.async_copy(ones_v, cnt_sh.at[lbls[b]], ssem[b], add=True)

        def wait_scatter(b):
            pltpu.make_async_copy(ones_v, cnt_sh.at[lbls[b]],
                                  ssem[b]).wait()

        issue_load(0, 0)
        issue_load(1, 1)

        def chunk_body(j, b):
            wait_load(b)
            issue_scatter(b)
            b2 = (b + 2) % 4

            @pl.when(j >= 2)
            def _():
                wait_scatter(b2)

            @pl.when(j + 2 < _CNCHUNKS)
            def _():
                issue_load(j + 2, b2)

        def gbody(g, _):
            for i in range(4):
                chunk_body(4 * g + i, i)
            return 0
        lax.fori_loop(0, (_CNCHUNKS - 1) // 4, gbody, 0)

        wait_load(0)
        issue_scatter(0)

        for b in (2, 3, 0):
            wait_scatter(b)

        plsc.subcore_barrier()

        _over_stripe(tile, lambda r0: pltpu.sync_copy(
            cnt_sh.at[pl.ds(r0, _ZROWS)],
            cnt_out.at[pl.ds(core * _C + r0, _ZROWS)]))

    return sc_kernel(embeddings, labels)


def _finalize_body(cnta_ref, cntb_ref, sum_ref, sq_ref, mean_ref, std_ref,
                   om_ref, os_ref):
    n = cnta_ref[:, 0:1] + cntb_ref[:, 0:1]
    safe_n = jnp.maximum(n, 1.0)
    mean = sum_ref[...] / safe_n
    denom = jnp.maximum(n - 1.0, 1.0)
    var = (sq_ref[...] - safe_n * mean * mean) / denom
    std = jnp.sqrt(jnp.maximum(var, 0.0))
    present = n > 0.0
    om_ref[...] = jnp.where(present, mean, mean_ref[...])
    os_ref[...] = jnp.where(present, std, std_ref[...])


def kernel(embeddings, labels, means, stds):
    acc = _sc_sums(embeddings, labels)
    cnt2 = _sc_counts(embeddings, labels)
    sums = acc[:_C]
    sumsq = acc[_C:]

    blk = 1000
    spec = pl.BlockSpec((blk, _D), lambda i: (i, 0))
    new_means, new_stds = pl.pallas_call(
        _finalize_body,
        grid=(_C // blk,),
        in_specs=[spec] * 6,
        out_specs=[spec, spec],
        out_shape=[jax.ShapeDtypeStruct((_C, _D), jnp.float32)] * 2,
    )(cnt2[:_C], cnt2[_C:], sums, sumsq, means, stds)
    return new_means, new_stds


# speedup vs baseline: 6.3228x; 1.0005x over previous
import functools

import jax
import jax.numpy as jnp
from jax import lax
from jax.experimental import pallas as pl
from jax.experimental.pallas import tpu as pltpu
from jax.experimental.pallas import tpu_sc as plsc

_N = 320000
_D = 128
_C = 10000

_NS = 16
_L = 16

_CHUNK = 80
_ROWS_PER_TILE = _N // _NS
_NCHUNKS = _ROWS_PER_TILE // _CHUNK
_CROWS_PER_TILE = _N // 2 // _NS
_CNCHUNKS = _CROWS_PER_TILE // _CHUNK

_CSTRIPE = 640
_ZROWS = 80
_NZ_FULL = _CSTRIPE // _ZROWS
_NZ_LAST = (_C - 15 * _CSTRIPE) // _ZROWS


def _over_stripe(tile, fn):
    @pl.when(tile < _NS - 1)
    def _():
        for z in range(_NZ_FULL):
            fn(tile * _CSTRIPE + z * _ZROWS)

    @pl.when(tile == _NS - 1)
    def _():
        for z in range(_NZ_LAST):
            fn((_NS - 1) * _CSTRIPE + z * _ZROWS)


def _zero_arith(ref, seed_ref):
    def zbody(r, _):
        for c in range(_D // _L):
            v = seed_ref[r, pl.ds(c * _L, _L)]
            ref[r, pl.ds(c * _L, _L)] = v * 0.0
        return 0
    lax.fori_loop(0, _CHUNK, zbody, 0)


def _sc_sums(embeddings, labels):
    mesh = plsc.VectorSubcoreMesh(core_axis_name="c", subcore_axis_name="s")

    @functools.partial(
        pl.kernel,
        mesh=mesh,
        out_type=jax.ShapeDtypeStruct((2 * _C, _D), jnp.float32),
        scratch_types=[
            pltpu.VMEM((_CHUNK, _D), jnp.float32),
            pltpu.VMEM((_CHUNK, _D), jnp.float32),
            pltpu.VMEM((_CHUNK, _D), jnp.float32),
            pltpu.VMEM((_CHUNK, _D), jnp.float32),
            pltpu.VMEM((_CHUNK,), jnp.int32),
            pltpu.VMEM((_CHUNK,), jnp.int32),
            pltpu.VMEM((_CHUNK,), jnp.int32),
            pltpu.VMEM((_CHUNK,), jnp.int32),
            pltpu.SemaphoreType.DMA,
            pltpu.SemaphoreType.DMA,
            pltpu.SemaphoreType.DMA,
            pltpu.SemaphoreType.DMA,
            pltpu.SemaphoreType.DMA,
            pltpu.SemaphoreType.DMA,
            pltpu.SemaphoreType.DMA,
            pltpu.SemaphoreType.DMA,
            pltpu.VMEM_SHARED((_C, _D), jnp.float32),
        ],
    )
    def sc_kernel(emb_hbm, lbl_hbm, acc_out,
                  row0, row1, row2, row3, lb0, lb1, lb2, lb3,
                  ls0, ls1, ls2, ls3, ss0, ss1, ss2, ss3, acc_sh):
        rows = [row0, row1, row2, row3]
        lbls = [lb0, lb1, lb2, lb3]
        lsem = [ls0, ls1, ls2, ls3]
        ssem = [ss0, ss1, ss2, ss3]

        core = lax.axis_index("c")
        tile = lax.axis_index("s")
        fcore = lax.convert_element_type(core, jnp.float32)
        base0 = tile * _ROWS_PER_TILE

        pltpu.sync_copy(emb_hbm.at[pl.ds(0, _CHUNK)], row0)
        _zero_arith(row0, row0)
        _over_stripe(tile, lambda r0: pltpu.async_copy(
            row0, acc_sh.at[pl.ds(r0, _ZROWS)], ls0))
        _over_stripe(tile, lambda r0: pltpu.make_async_copy(
            row0, acc_sh.at[pl.ds(r0, _ZROWS)], ls0).wait())

        plsc.subcore_barrier()

        def issue_loads(j, b):
            base = base0 + j * _CHUNK
            pltpu.async_copy(lbl_hbm.at[pl.ds(base, _CHUNK)], lbls[b],
                             lsem[b])
            pltpu.async_copy(emb_hbm.at[pl.ds(base, _CHUNK)], rows[b],
                             lsem[b])

        def wait_loads(b):
            pltpu.make_async_copy(lbl_hbm.at[pl.ds(0, _CHUNK)], lbls[b],
                                  lsem[b]).wait()
            pltpu.make_async_copy(emb_hbm.at[pl.ds(0, _CHUNK)], rows[b],
                                  lsem[b]).wait()

        def issue_scatter(b):
            pltpu.async_copy(rows[b], acc_sh.at[lbls[b]], ssem[b], add=True)

        def wait_scatter(b):
            pltpu.make_async_copy(rows[b], acc_sh.at[lbls[b]],
                                  ssem[b]).wait()

        def squares(b):
            def sq(r, _):
                for c in range(_D // _L):
                    v = rows[b][r, pl.ds(c * _L, _L)]
                    rows[b][r, pl.ds(c * _L, _L)] = v * (fcore * v +
                                                         (1.0 - fcore))
                return 0
            lax.fori_loop(0, _CHUNK, sq, 0)

        issue_loads(0, 0)
        issue_loads(1, 1)

        def chunk_body(j, b):
            wait_loads(b)
            squares(b)
            issue_scatter(b)
            b2 = (b + 2) % 4

            @pl.when(j >= 2)
            def _():
                wait_scatter(b2)
            issue_loads(j + 2, b2)

        def gbody(g, _):
            for i in range(4):
                chunk_body(4 * g + i, i)
            return 0
        lax.fori_loop(0, (_NCHUNKS - 2) // 4, gbody, 0)

        for (j, b) in ((_NCHUNKS - 2, 0), (_NCHUNKS - 1, 1)):
            wait_loads(b)
            squares(b)
            issue_scatter(b)

        for b in range(4):
            wait_scatter(b)

        plsc.subcore_barrier()

        _over_stripe(tile, lambda r0: pltpu.async_copy(
            acc_sh.at[pl.ds(r0, _ZROWS)],
            acc_out.at[pl.ds(core * _C + r0, _ZROWS)], ls0))
        _over_stripe(tile, lambda r0: pltpu.make_async_copy(
            acc_sh.at[pl.ds(r0, _ZROWS)],
            acc_out.at[pl.ds(core * _C + r0, _ZROWS)], ls0).wait())

    return sc_kernel(embeddings, labels)


def _sc_counts(embeddings, labels):
    mesh = plsc.VectorSubcoreMesh(core_axis_name="c", subcore_axis_name="s")

    @functools.partial(
        pl.kernel,
        mesh=mesh,
        out_type=jax.ShapeDtypeStruct((2 * _C, _D), jnp.float32),
        scratch_types=[
            pltpu.VMEM((_CHUNK, _D), jnp.float32),
            pltpu.VMEM((_CHUNK,), jnp.int32),
            pltpu.VMEM((_CHUNK,), jnp.int32),
            pltpu.VMEM((_CHUNK,), jnp.int32),
            pltpu.VMEM((_CHUNK,), jnp.int32),
            pltpu.SemaphoreType.DMA,
            pltpu.SemaphoreType.DMA,
            pltpu.SemaphoreType.DMA,
            pltpu.SemaphoreType.DMA,
            pltpu.SemaphoreType.DMA,
            pltpu.SemaphoreType.DMA,
            pltpu.SemaphoreType.DMA,
            pltpu.SemaphoreType.DMA,
            pltpu.VMEM_SHARED((_C, _D), jnp.float32),
        ],
    )
    def sc_kernel(emb_hbm, lbl_hbm, cnt_out, ones_v,
                  lb0, lb1, lb2, lb3, ls0, ls1, ls2, ls3,
                  ss0, ss1, ss2, ss3, cnt_sh):
        lbls = [lb0, lb1, lb2, lb3]
        lsem = [ls0, ls1, ls2, ls3]
        ssem = [ss0, ss1, ss2, ss3]

        core = lax.axis_index("c")
        tile = lax.axis_index("s")
        base0 = (core * _NS + tile) * _CROWS_PER_TILE

        pltpu.sync_copy(emb_hbm.at[pl.ds(0, _CHUNK)], ones_v)
        _zero_arith(ones_v, ones_v)
        _over_stripe(tile, lambda r0: pltpu.async_copy(
            ones_v, cnt_sh.at[pl.ds(r0, _ZROWS)], ls0))
        _over_stripe(tile, lambda r0: pltpu.make_async_copy(
            ones_v, cnt_sh.at[pl.ds(r0, _ZROWS)], ls0).wait())

        def obody(r, _):
            for c in range(_D // _L):
                z = ones_v[r, pl.ds(c * _L, _L)]
                ones_v[r, pl.ds(c * _L, _L)] = z + 1.0
            return 0
        lax.fori_loop(0, _CHUNK, obody, 0)

        plsc.subcore_barrier()

        def issue_load(j, b):
            base = base0 + j * _CHUNK
            pltpu.async_copy(lbl_hbm.at[pl.ds(base, _CHUNK)], lbls[b],
                             lsem[b])

        def wait_load(b):
            pltpu.make_async_copy(lbl_hbm.at[pl.ds(0, _CHUNK)], lbls[b],
                                  lsem[b]).wait()

        def issue_scatter(b):
            pltpu.async_copy(ones_v, cnt_sh.at[lbls[b]], ssem[b], add=True)

        def wait_scatter(b):
            pltpu.make_async_copy(ones_v, cnt_sh.at[lbls[b]],
                                  ssem[b]).wait()

        issue_load(0, 0)
        issue_load(1, 1)

        def chunk_body(j, b):
            wait_load(b)
            issue_scatter(b)
            b2 = (b + 2) % 4

            @pl.when(j >= 2)
            def _():
                wait_scatter(b2)

            @pl.when(j + 2 < _CNCHUNKS)
            def _():
                issue_load(j + 2, b2)

        def gbody(g, _):
            for i in range(4):
                chunk_body(4 * g + i, i)
            return 0
        lax.fori_loop(0, (_CNCHUNKS - 1) // 4, gbody, 0)

        wait_load(0)
        issue_scatter(0)

        for b in (2, 3, 0):
            wait_scatter(b)

        plsc.subcore_barrier()

        _over_stripe(tile, lambda r0: pltpu.async_copy(
            cnt_sh.at[pl.ds(r0, _ZROWS)],
            cnt_out.at[pl.ds(core * _C + r0, _ZROWS)], ls0))
        _over_stripe(tile, lambda r0: pltpu.make_async_copy(
            cnt_sh.at[pl.ds(r0, _ZROWS)],
            cnt_out.at[pl.ds(core * _C + r0, _ZROWS)], ls0).wait())

    return sc_kernel(embeddings, labels)


def _finalize_body(cnta_ref, cntb_ref, sum_ref, sq_ref, mean_ref, std_ref,
                   om_ref, os_ref):
    n = cnta_ref[:, 0:1] + cntb_ref[:, 0:1]
    safe_n = jnp.maximum(n, 1.0)
    mean = sum_ref[...] / safe_n
    denom = jnp.maximum(n - 1.0, 1.0)
    var = (sq_ref[...] - safe_n * mean * mean) / denom
    std = jnp.sqrt(jnp.maximum(var, 0.0))
    present = n > 0.0
    om_ref[...] = jnp.where(present, mean, mean_ref[...])
    os_ref[...] = jnp.where(present, std, std_ref[...])


def kernel(embeddings, labels, means, stds):
    acc = _sc_sums(embeddings, labels)
    cnt2 = _sc_counts(embeddings, labels)
    sums = acc[:_C]
    sumsq = acc[_C:]

    blk = 1000
    spec = pl.BlockSpec((blk, _D), lambda i: (i, 0))
    new_means, new_stds = pl.pallas_call(
        _finalize_body,
        grid=(_C // blk,),
        in_specs=[spec] * 6,
        out_specs=[spec, spec],
        out_shape=[jax.ShapeDtypeStruct((_C, _D), jnp.float32)] * 2,
    )(cnt2[:_C], cnt2[_C:], sums, sumsq, means, stds)
    return new_means, new_stds
